# baseline (speedup 1.0000x reference)
import jax
import jax.numpy as jnp
from jax import lax
from jax.experimental import pallas as pl
from jax.experimental.pallas import tpu as pltpu
from jax.experimental.pallas import tpu_sc as plsc

_N_WIDTH = 32
_N_NODES = 257
_N_SAMPLES = 4096
_N_WORKERS = 32
_SPW = _N_SAMPLES // _N_WORKERS
_RPW = 8
_NCHUNKS = _N_SAMPLES // 16
_PB = 16

_F32 = jnp.float32
_I32 = jnp.int32


def _sc_body(x_hbm, ddphi_hbm, x_v, stag0, stag1, stag2,
             sem0, sem1, sem2):
    wid = lax.axis_index("s") * 2 + lax.axis_index("c")
    base = wid * _SPW

    pltpu.sync_copy(x_hbm, x_v)

    fzero = jnp.zeros((16,), _F32)

    def nodes(xb):
        xs = xb * 256.0
        eli = jnp.clip((xs * 0.5).astype(_I32), 0, 127)
        return eli * 2

    stags = ((stag0, sem0), (stag1, sem1), (stag2, sem2))

    def build_and_fire(rowp, b):
        stag, sem = stags[b]

        @pl.loop(0, _NCHUNKS, unroll=4)
        def _chunks(c):
            nli = nodes(x_v[pl.ds(c * 16, 16)])
            m0 = nli == rowp
            m1 = nli == rowp - 1
            m2 = nli == rowp - 2
            v0 = jnp.full((16,), 65536.0, _F32)
            v1 = jnp.full((16,), -131072.0, _F32)
            val = (jnp.where(m0, v0, fzero) + jnp.where(m1, v1, fzero)
                   + jnp.where(m2, v0, fzero))
            off = c * 16
            for r in range(8):
                stag[r, pl.ds(off, 16)] = val

        for h in range(4):
            pltpu.async_copy(stag, ddphi_hbm.at[rowp, pl.ds(h * 8, 8)], sem)

    def drain(b):
        stag, sem = stags[b]
        for h in range(4):
            pltpu.make_async_copy(stag, ddphi_hbm.at[0, pl.ds(h * 8, 8)],
                                  sem).wait()

    for r in range(_RPW):
        b = r % 3
        if r >= 3:
            drain(b)
        build_and_fire(wid * _RPW + r, b)

    drain(0)

    @pl.loop(0, _SPW // 16)
    def _c256(c):
        nli = nodes(x_v[pl.ds(base + c * 16, 16)])
        v0 = jnp.full((16,), 65536.0, _F32)
        v1 = jnp.full((16,), -131072.0, _F32)
        val = (jnp.where(nli == _N_NODES - 1, v0, fzero)
               + jnp.where(nli == _N_NODES - 2, v1, fzero)
               + jnp.where(nli == _N_NODES - 3, v0, fzero))
        for r in range(8):
            stag0[r, pl.ds(c * 16, 16)] = val

    src256 = stag0.at[:, pl.ds(0, _SPW)]
    for h in range(4):
        pltpu.async_copy(
            src256,
            ddphi_hbm.at[_N_NODES - 1, pl.ds(h * 8, 8), pl.ds(base, _SPW)],
            sem0)

    for h in range(4):
        pltpu.make_async_copy(
            src256,
            ddphi_hbm.at[_N_NODES - 1, pl.ds(h * 8, 8), pl.ds(0, _SPW)],
            sem0).wait()
    drain(1)
    drain(2)


def _tc_body(x_ref, w_ref, phi_ref, dphi_ref, t_ref, dt_ref, ddt_ref):
    g = pl.program_id(0)
    x = x_ref[...]
    xs = x * 256.0
    nlf = jnp.clip(jnp.floor(xs * 0.5), 0.0, 127.0) * 2.0
    t = xs - nlf - 1.0
    p0 = 0.5 * t * (t - 1.0)
    p1 = 1.0 - t * t
    p2 = 0.5 * t * (t + 1.0)
    d0 = (t - 0.5) * 256.0
    d1 = t * -512.0
    d2 = (t + 0.5) * 256.0
    nli = nlf.astype(_I32)
    prow = g * _PB + lax.broadcasted_iota(_I32, (_PB, _N_SAMPLES), 0)
    rel = prow - nli[None, :]
    m0 = rel == 0
    m1 = rel == 1
    m2 = rel == 2
    zero = jnp.zeros((), _F32)
    phi_row = jnp.where(m0, p0[None, :],
                        jnp.where(m1, p1[None, :],
                                  jnp.where(m2, p2[None, :], zero)))
    dphi_row = jnp.where(m0, d0[None, :],
                         jnp.where(m1, d1[None, :],
                                   jnp.where(m2, d2[None, :], zero)))
    ddphi_row = (jnp.where(m0, 65536.0, zero) + jnp.where(m1, -131072.0, zero)
                 + jnp.where(m2, 65536.0, zero))
    shp = (_PB, _N_WIDTH, _N_SAMPLES)
    phi_ref[...] = jnp.broadcast_to(phi_row[:, None, :], shp)
    dphi_ref[...] = jnp.broadcast_to(dphi_row[:, None, :], shp)

    wb = w_ref[...]
    col = g * _PB + lax.broadcasted_iota(_I32, (_PB, _N_WIDTH), 0)
    wb = jnp.where(col < _N_NODES, wb, zero)
    dn = (((0,), (0,)), ((), ()))
    pt = lax.dot_general(wb, phi_row, dn, preferred_element_type=_F32)
    pdt = lax.dot_general(wb, dphi_row, dn, preferred_element_type=_F32)
    pddt = lax.dot_general(wb, ddphi_row, dn, preferred_element_type=_F32)

    @pl.when(g == 0)
    def _init():
        t_ref[...] = pt
        dt_ref[...] = pdt
        ddt_ref[...] = pddt

    @pl.when(g > 0)
    def _acc():
        t_ref[...] = t_ref[...] + pt
        dt_ref[...] = dt_ref[...] + pdt
        ddt_ref[...] = ddt_ref[...] + pddt


@jax.jit
def kernel(x, weight):
    mesh = plsc.VectorSubcoreMesh(core_axis_name="c", subcore_axis_name="s")
    big = jax.ShapeDtypeStruct((_N_NODES, _N_WIDTH, _N_SAMPLES), _F32)
    small = jax.ShapeDtypeStruct((_N_WIDTH, _N_SAMPLES), _F32)
    sc_fn = pl.kernel(
        _sc_body,
        out_type=big,
        mesh=mesh,
        compiler_params=pltpu.CompilerParams(needs_layout_passes=False),
        scratch_types=[
            pltpu.VMEM((_N_SAMPLES,), _F32),
            pltpu.VMEM((8, _N_SAMPLES), _F32),
            pltpu.VMEM((8, _N_SAMPLES), _F32),
            pltpu.VMEM((8, _N_SAMPLES), _F32),
            pltpu.SemaphoreType.DMA,
            pltpu.SemaphoreType.DMA,
            pltpu.SemaphoreType.DMA,
        ],
    )
    ddphi = sc_fn(x)

    nsteps = (_N_NODES + _PB - 1) // _PB
    phi, dphi, t, dt, ddt = pl.pallas_call(
        _tc_body,
        grid=(nsteps,),
        in_specs=[
            pl.BlockSpec((_N_SAMPLES,), lambda g: (0,)),
            pl.BlockSpec((_PB, _N_WIDTH), lambda g: (g, 0)),
        ],
        out_specs=(
            pl.BlockSpec((_PB, _N_WIDTH, _N_SAMPLES), lambda g: (g, 0, 0)),
            pl.BlockSpec((_PB, _N_WIDTH, _N_SAMPLES), lambda g: (g, 0, 0)),
            pl.BlockSpec((_N_WIDTH, _N_SAMPLES), lambda g: (0, 0)),
            pl.BlockSpec((_N_WIDTH, _N_SAMPLES), lambda g: (0, 0)),
            pl.BlockSpec((_N_WIDTH, _N_SAMPLES), lambda g: (0, 0)),
        ),
        out_shape=(big, big, small, small, small),
    )(x, weight.T)

    tr3 = lambda a: jnp.transpose(a, (2, 1, 0))
    return (t.T, dt.T, ddt.T, tr3(phi), tr3(dphi), tr3(ddphi))
